# Initial kernel scaffold; baseline (speedup 1.0000x reference)
import jax
import jax.numpy as jnp
from jax.experimental import pallas as pl


def kernel(x_user, x_item, ei_ui, ei_iu, W_in, b_in, kqva_W, kqva_b, rel_a, rel_m, rel_p, skip, ln_g, ln_b, W_out, b_out):
    raise NotImplementedError("write your pallas kernel here")


# speedup vs baseline: 14.6391x; 14.6391x over previous
import functools
import math

import jax
import jax.numpy as jnp
from jax import lax
from jax.experimental import pallas as pl
from jax.experimental.pallas import tpu as pltpu
from jax.experimental.pallas import tpu_sc as plsc

NC = 2
NS = 16
LN = 16
C = 512
FB = 250


def _mesh():
    return plsc.VectorSubcoreMesh(
        core_axis_name="c", subcore_axis_name="s", num_cores=NC, num_subcores=NS
    )


def _passa_body(nchunk, iters,
                q0, k0, s0, d0, q1, k1, s1, d1,
                a0, a1, tmax,
                idxd, idxs, qg, kg, astage, mst):
    cc = lax.axis_index("c")
    ss = lax.axis_index("s")
    wid = ss * NC + cc
    lane = lax.iota(jnp.int32, LN)
    G = C // LN
    for et, (q_h, k_h, s_h, d_h, a_h) in enumerate(
            ((q0, k0, s0, d0, a0), (q1, k1, s1, d1, a1))):
        def chunk(kk, m):
            cid = jnp.minimum(kk * (NC * NS) + wid, nchunk - 1)
            off = pl.multiple_of(cid * C, 8)
            for j in range(C // 128):
                pltpu.sync_copy(d_h.at[pl.ds(off + j * 128, 128)], idxd.at[j])
                pltpu.sync_copy(s_h.at[pl.ds(off + j * 128, 128)], idxs.at[j])
            for j in range(C // 128):
                pltpu.sync_copy(q_h.at[idxd.at[j]], qg.at[pl.ds(j * 128, 128), :])
                pltpu.sync_copy(k_h.at[idxs.at[j]], kg.at[pl.ds(j * 128, 128), :])
            def grp(g, m):
                rows = g * LN + lane
                out = []
                for h in range(4):
                    acc = jnp.zeros((LN,), jnp.float32)
                    for dd in range(16):
                        col = jnp.full((LN,), h * 16 + dd, jnp.int32)
                        vq = plsc.load_gather(qg, [rows, col])
                        vk = plsc.load_gather(kg, [rows, col])
                        acc = acc + vq * vk
                    plsc.store_scatter(astage, [rows * 4 + h], acc)
                    out.append(jnp.maximum(m[h], acc))
                return tuple(out)

            m = lax.fori_loop(0, G, grp, m)
            pltpu.sync_copy(astage, a_h.at[pl.ds(off * 4, C * 4)])
            return m

        minit = tuple(jnp.full((LN,), -1e30, jnp.float32) for _ in range(4))
        m = lax.fori_loop(0, iters, chunk, minit)
        mv = jnp.full((LN,), -1e30, jnp.float32)
        for h in range(4):
            mv = jnp.where(lane == h, jnp.max(m[h]), mv)
        mst[...] = mv
        pltpu.sync_copy(mst, tmax.at[et, wid])


@functools.partial(jax.jit, static_argnames=("ep",))
def _passa(ep, q0, k0, s0, d0, q1, k1, s1, d1):
    nchunk = ep // C
    iters = -(-nchunk // (NC * NS))
    kern = pl.kernel(
        functools.partial(_passa_body, nchunk, iters),
        out_type=(
            jax.ShapeDtypeStruct((ep * 4,), jnp.float32),
            jax.ShapeDtypeStruct((ep * 4,), jnp.float32),
            jax.ShapeDtypeStruct((2, NC * NS, LN), jnp.float32),
        ),
        mesh=_mesh(),
        scratch_types=[
            pltpu.VMEM((C // 128, 128), jnp.int32),
            pltpu.VMEM((C // 128, 128), jnp.int32),
            pltpu.VMEM((C, 64), jnp.float32),
            pltpu.VMEM((C, 64), jnp.float32),
            pltpu.VMEM((C * 4,), jnp.float32),
            pltpu.VMEM((LN,), jnp.float32),
        ],
        compiler_params=pltpu.CompilerParams(
            needs_layout_passes=False, use_tc_tiling_on_sc=False),
    )
    return kern(q0, k0, s0, d0, q1, k1, s1, d1)


def _passb_body(n_nodes, e_real, nchunk, rows_sp, rhalf,
                alpha, tmax, s_e, d_e, ve,
                numer, den,
                idxd, idxs, ldst, vg, al_v, msg, sem, agg_sp):
    cc = lax.axis_index("c")
    sid = lax.axis_index("s")
    lane = lax.iota(jnp.int32, LN)
    quarter = n_nodes // 4
    base_g = (rhalf * NC + cc) * quarter
    base_l = cc * quarter
    trash = quarter
    G = C // LN
    zz = jnp.zeros((LN,), jnp.float32)

    pltpu.async_copy(tmax, msg.at[pl.ds(0, NC * NS), pl.ds(0, LN)], sem).wait()
    def mred(i, m):
        return jnp.maximum(m, msg[i, pl.ds(0, LN)])
    m = lax.fori_loop(0, NC * NS, mred, jnp.full((LN,), -1e30, jnp.float32))
    lm = lane % 4
    gt = jnp.where(lm == 0, m[0],
                   jnp.where(lm == 1, m[1],
                             jnp.where(lm == 2, m[2], m[3])))

    def zrow(e, _):
        for kcol in (0, 16, 32, 48, 56):
            msg[e, pl.ds(kcol, LN)] = zz
        return 0
    lax.fori_loop(0, C, zrow, 0)

    nz = -(-rows_sp // 128)
    def zchunk(kk, _):
        rr = jnp.minimum(kk * NS + sid, nz - 1)
        r0 = jnp.minimum(rr * 128, rows_sp - 128)
        pltpu.sync_copy(msg.at[pl.ds(0, 128), :],
                        agg_sp.at[pl.ds(r0, 128), :])
        return 0
    lax.fori_loop(0, -(-nz // NS), zchunk, 0)
    plsc.subcore_barrier()

    def chunk(kk, _):
        cidr = kk * NS + sid
        dup = cidr >= nchunk
        cid = jnp.minimum(cidr, nchunk - 1)
        off = pl.multiple_of(cid * C, 8)
        hs = []
        for j in range(C // 128):
            hs.append(pltpu.async_copy(
                d_e.at[pl.ds(off + j * 128, 128)], idxd.at[j], sem))
            hs.append(pltpu.async_copy(
                s_e.at[pl.ds(off + j * 128, 128)], idxs.at[j], sem))
        hs.append(pltpu.async_copy(alpha.at[pl.ds(off * 4, C * 4)], al_v, sem))
        for h in hs:
            h.wait()
        hs = []
        for j in range(C // 128):
            hs.append(pltpu.async_copy(
                ve.at[idxs.at[j]], vg.at[pl.ds(j * 128, 128), :], sem))
        for h in hs:
            h.wait()

        def exg(i, _):
            a = al_v[pl.ds(i * LN, LN)]
            al_v[pl.ds(i * LN, LN)] = jnp.exp(a - gt)
            return 0
        lax.fori_loop(0, C * 4 // LN, exg, 0)

        for j in range(C // 128):
            for i in range(128 // LN):
                dd = idxd[j, pl.ds(i * LN, LN)]
                loc = dd - base_g
                eix = off + j * 128 + i * LN + lane
                ok = (loc >= 0) & (loc < quarter) & (eix < e_real) & jnp.logical_not(dup)
                ldst[j, pl.ds(i * LN, LN)] = jnp.where(ok, loc, trash)

        def grp(g, _):
            rows = g * LN + lane
            for h in range(4):
                exh = plsc.load_gather(al_v, [rows * 4 + h])
                plsc.store_scatter(msg, [rows, jnp.full((LN,), 64 + h, jnp.int32)], exh)
                for dd in range(16):
                    col = jnp.full((LN,), h * 16 + dd, jnp.int32)
                    vv = plsc.load_gather(vg, [rows, col])
                    plsc.store_scatter(msg, [rows, col], vv * exh)
            return 0
        lax.fori_loop(0, G, grp, 0)

        hs = []
        for j in range(C // 128):
            hs.append(pltpu.async_copy(
                msg.at[pl.ds(j * 128, 128), :],
                agg_sp.at[ldst.at[j]], sem, add=True))
        for h in hs:
            h.wait()
        return 0

    lax.fori_loop(0, -(-nchunk // NS), chunk, 0)
    plsc.subcore_barrier()

    nf = quarter // FB
    def flush(kk, _):
        rr = jnp.minimum(kk * NS + sid, nf - 1)
        r0 = rr * FB
        pltpu.sync_copy(agg_sp.at[pl.ds(r0, FB), pl.ds(0, 64)],
                        numer.at[pl.ds(base_l + r0, FB), :])
        pltpu.sync_copy(agg_sp.at[pl.ds(r0, FB), pl.ds(64, 8)],
                        den.at[pl.ds(base_l + r0, FB), :])
        return 0
    lax.fori_loop(0, -(-nf // NS), flush, 0)


def _passb_half(n_nodes, e_real, ep, rhalf, alpha, tmax, s_e, d_e, ve):
    nchunk = ep // C
    rows_sp = n_nodes // 4 + 8
    kern = pl.kernel(
        functools.partial(_passb_body, n_nodes, e_real, nchunk, rows_sp, rhalf),
        out_type=(
            jax.ShapeDtypeStruct((n_nodes // 2, 64), jnp.float32),
            jax.ShapeDtypeStruct((n_nodes // 2, 8), jnp.float32),
        ),
        mesh=_mesh(),
        scratch_types=[
            pltpu.VMEM((C // 128, 128), jnp.int32),
            pltpu.VMEM((C // 128, 128), jnp.int32),
            pltpu.VMEM((C // 128, 128), jnp.int32),
            pltpu.VMEM((C, 64), jnp.float32),
            pltpu.VMEM((C * 4,), jnp.float32),
            pltpu.VMEM((C, 72), jnp.float32),
            pltpu.SemaphoreType.DMA,
            pltpu.VMEM_SHARED((rows_sp, 72), jnp.float32),
        ],
        compiler_params=pltpu.CompilerParams(
            needs_layout_passes=False, use_tc_tiling_on_sc=False),
    )
    return kern(alpha, tmax, s_e, d_e, ve)


@functools.partial(jax.jit, static_argnames=("n_nodes", "e_real", "ep"))
def _passb(n_nodes, e_real, ep, alpha, tmax, s_e, d_e, ve):
    n0, d0 = _passb_half(n_nodes, e_real, ep, 0, alpha, tmax, s_e, d_e, ve)
    n1, d1 = _passb_half(n_nodes, e_real, ep, 1, alpha, tmax, s_e, d_e, ve)
    return (jnp.concatenate([n0, n1], axis=0),
            jnp.concatenate([d0, d1], axis=0))


def _block_rows(m):
    for bm in (2000, 1000, 400, 200, 80, 40, 16, 8):
        if m % bm == 0:
            return bm
    return m


def _mm_body(x_ref, w_ref, b_ref, o_ref):
    o_ref[...] = (
        jnp.dot(x_ref[...], w_ref[...], preferred_element_type=jnp.float32)
        + b_ref[...]
    )


def _mm_bias(x, w, b):
    m, kd = x.shape
    n = w.shape[1]
    bm = _block_rows(m)
    return pl.pallas_call(
        _mm_body,
        grid=(m // bm,),
        in_specs=[
            pl.BlockSpec((bm, kd), lambda i: (i, 0)),
            pl.BlockSpec((kd, n), lambda i: (0, 0)),
            pl.BlockSpec((1, n), lambda i: (0, 0)),
        ],
        out_specs=pl.BlockSpec((bm, n), lambda i: (i, 0)),
        out_shape=jax.ShapeDtypeStruct((m, n), jnp.float32),
    )(x, w, b.reshape(1, n))


def _kqv_body(h_ref, w_ref, b_ref, k_ref, q_ref, v_ref):
    hh = h_ref[...]
    for i, o_ref in enumerate((k_ref, q_ref, v_ref)):
        o_ref[...] = (
            jnp.dot(hh, w_ref[i], preferred_element_type=jnp.float32)
            + b_ref[0, i]
        )


def _kqv(h, w3, b3):
    m = h.shape[0]
    bm = _block_rows(m)
    out = jax.ShapeDtypeStruct((m, 64), jnp.float32)
    return pl.pallas_call(
        _kqv_body,
        grid=(m // bm,),
        in_specs=[
            pl.BlockSpec((bm, 64), lambda i: (i, 0)),
            pl.BlockSpec((3, 64, 64), lambda i: (0, 0, 0)),
            pl.BlockSpec((1, 3, 64), lambda i: (0, 0, 0)),
        ],
        out_specs=[pl.BlockSpec((bm, 64), lambda i: (i, 0))] * 3,
        out_shape=[out, out, out],
    )(h, w3, b3.reshape(1, 3, 64))


def _post_body(numer_ref, den_ref, h_ref, w_ref, p_ref, o_ref):
    s = (lax.broadcasted_iota(jnp.int32, (8, 64), 1) // 16
         == lax.broadcasted_iota(jnp.int32, (8, 64), 0)).astype(jnp.float32)
    den_b = jnp.dot(den_ref[...], s, preferred_element_type=jnp.float32)
    agg = numer_ref[...] / (den_b + 1e-16)
    o = jax.nn.gelu(agg)
    o = (jnp.dot(o, w_ref[...], preferred_element_type=jnp.float32)
         + p_ref[0:1, :] + p_ref[1:2, :] * h_ref[...])
    mu = jnp.mean(o, axis=1, keepdims=True)
    var = jnp.mean((o - mu) ** 2, axis=1, keepdims=True)
    o = (o - mu) / jnp.sqrt(var + 1e-5) * p_ref[2:3, :] + p_ref[3:4, :]
    o_ref[...] = jax.nn.gelu(o)


def _post(numer, den, h, w_eff, pvec):
    m = h.shape[0]
    bm = _block_rows(m)
    return pl.pallas_call(
        _post_body,
        grid=(m // bm,),
        in_specs=[
            pl.BlockSpec((bm, 64), lambda i: (i, 0)),
            pl.BlockSpec((bm, 8), lambda i: (i, 0)),
            pl.BlockSpec((bm, 64), lambda i: (i, 0)),
            pl.BlockSpec((64, 64), lambda i: (0, 0)),
            pl.BlockSpec((4, 64), lambda i: (0, 0)),
        ],
        out_specs=pl.BlockSpec((bm, 64), lambda i: (i, 0)),
        out_shape=jax.ShapeDtypeStruct((m, 64), jnp.float32),
    )(numer, den, h, w_eff, pvec)


def kernel(x_user, x_item, ei_ui, ei_iu, W_in, b_in, kqva_W, kqva_b,
           rel_a, rel_m, rel_p, skip, ln_g, ln_b, W_out, b_out):
    n_u, n_i = x_user.shape[0], x_item.shape[0]
    e = ei_ui.shape[1]
    ep = -(-e // C) * C
    L = kqva_W.shape[0]

    pad = jnp.zeros((ep - e,), jnp.int32) if ep != e else None
    def _pad(a):
        a = a.astype(jnp.int32)
        return a if pad is None else jnp.concatenate([a, pad])
    s0, d0 = _pad(ei_ui[0]), _pad(ei_ui[1])
    s1, d1 = _pad(ei_iu[0]), _pad(ei_iu[1])

    h = [_mm_bias(x_user, W_in[0], b_in[0]), _mm_bias(x_item, W_in[1], b_in[1])]

    for l in range(L):
        kqv = []
        for t in range(2):
            A, M = rel_a[l, t], rel_m[l, t]
            scale = jnp.repeat(rel_p[l, t] / 4.0, 16)
            wk = jnp.einsum("dhe,hef->dhf",
                            kqva_W[l, 0, t].reshape(64, 4, 16), A).reshape(64, 64)
            bk = jnp.einsum("he,hef->hf",
                            kqva_b[l, 0, t].reshape(4, 16), A).reshape(64)
            wv = jnp.einsum("dhe,hef->dhf",
                            kqva_W[l, 2, t].reshape(64, 4, 16), M).reshape(64, 64)
            bv = jnp.einsum("he,hef->hf",
                            kqva_b[l, 2, t].reshape(4, 16), M).reshape(64)
            w3 = jnp.stack([wk * scale[None, :], kqva_W[l, 1, t], wv])
            b3 = jnp.stack([bk * scale, kqva_b[l, 1, t], bv])
            kqv.append(_kqv(h[t], w3, b3))
        (k0, q0, v0), (k1, q1, v1) = kqv

        alpha0, alpha1, tmax = _passa(ep, q1, k0, s0, d0, q0, k1, s1, d1)
        numer1, den1 = _passb(n_i, e, ep, alpha0, tmax[0], s0, d0, v0)
        numer0, den0 = _passb(n_u, e, ep, alpha1, tmax[1], s1, d1, v1)

        new_h = []
        for t, (numer, den) in enumerate(((numer0, den0), (numer1, den1))):
            beta = jax.nn.sigmoid(skip[l, t])
            pvec = jnp.stack([
                kqva_b[l, 3, t] * beta,
                jnp.full((64,), 1.0, jnp.float32) * (1.0 - beta),
                ln_g[l, t],
                ln_b[l, t],
            ])
            new_h.append(_post(numer, den, h[t], kqva_W[l, 3, t] * beta, pvec))
        h = new_h

    return (_mm_bias(h[0], W_out[0], b_out[0]),
            _mm_bias(h[1], W_out[1], b_out[1]))


# speedup vs baseline: 16.5653x; 1.1316x over previous
import functools
import math

import jax
import jax.numpy as jnp
from jax import lax
from jax.experimental import pallas as pl
from jax.experimental.pallas import tpu as pltpu
from jax.experimental.pallas import tpu_sc as plsc

NC = 2
NS = 16
LN = 16
C = 256
FB = 250


def _mesh():
    return plsc.VectorSubcoreMesh(
        core_axis_name="c", subcore_axis_name="s", num_cores=NC, num_subcores=NS
    )


def _passa_body(nchunk, nitp,
                q0, k0, s0, d0, q1, k1, s1, d1,
                a0, a1, tmax,
                idxd0, idxs0, qg0, kg0, ast0,
                idxd1, idxs1, qg1, kg1, ast1, mst,
                semi0, semi1, semg0, semg1, semw0, semw1):
    cc = lax.axis_index("c")
    ss = lax.axis_index("s")
    wid = ss * NC + cc
    lane = lax.iota(jnp.int32, LN)
    G = C // LN
    NJ = C // 128
    bufs = ((idxd0, idxs0, qg0, kg0, ast0, semi0, semg0, semw0),
            (idxd1, idxs1, qg1, kg1, ast1, semi1, semg1, semw1))

    for et, (q_h, k_h, s_h, d_h, a_h) in enumerate(
            ((q0, k0, s0, d0, a0), (q1, k1, s1, d1, a1))):

        def off_of(kk):
            cid = jnp.minimum(kk * (NC * NS) + wid, nchunk - 1)
            return pl.multiple_of(cid * C, 8)

        def fire_idx(kk, b):
            off = off_of(kk)
            idxd, idxs, semi = bufs[b][0], bufs[b][1], bufs[b][5]
            for j in range(NJ):
                pltpu.async_copy(d_h.at[pl.ds(off + j * 128, 128)], idxd.at[j], semi)
                pltpu.async_copy(s_h.at[pl.ds(off + j * 128, 128)], idxs.at[j], semi)

        def wait_idx(b):
            idxd, idxs, semi = bufs[b][0], bufs[b][1], bufs[b][5]
            for j in range(NJ):
                pltpu.make_async_copy(d_h.at[pl.ds(0, 128)], idxd.at[j], semi).wait()
                pltpu.make_async_copy(s_h.at[pl.ds(0, 128)], idxs.at[j], semi).wait()

        def fire_gather(b):
            idxd, idxs, qg, kg, semg = (bufs[b][0], bufs[b][1], bufs[b][2],
                                        bufs[b][3], bufs[b][6])
            for j in range(NJ):
                pltpu.async_copy(q_h.at[idxd.at[j]], qg.at[pl.ds(j * 128, 128), :], semg)
                pltpu.async_copy(k_h.at[idxs.at[j]], kg.at[pl.ds(j * 128, 128), :], semg)

        def wait_gather(b):
            idxd, idxs, qg, kg, semg = (bufs[b][0], bufs[b][1], bufs[b][2],
                                        bufs[b][3], bufs[b][6])
            for j in range(NJ):
                pltpu.make_async_copy(q_h.at[idxd.at[j]], qg.at[pl.ds(j * 128, 128), :], semg).wait()
                pltpu.make_async_copy(k_h.at[idxs.at[j]], kg.at[pl.ds(j * 128, 128), :], semg).wait()

        def wait_w(b):
            ast, semw = bufs[b][4], bufs[b][7]
            pltpu.make_async_copy(a_h.at[pl.ds(0, C * 4)], ast, semw).wait()

        def compute(b, m):
            qg, kg, ast = bufs[b][2], bufs[b][3], bufs[b][4]

            def grp(g, m):
                rows = g * LN + lane
                out = []
                for h in range(4):
                    acc = jnp.zeros((LN,), jnp.float32)
                    for dd in range(16):
                        col = jnp.full((LN,), h * 16 + dd, jnp.int32)
                        vq = plsc.load_gather(qg, [rows, col])
                        vk = plsc.load_gather(kg, [rows, col])
                        acc = acc + vq * vk
                    plsc.store_scatter(ast, [rows * 4 + h], acc)
                    out.append(jnp.maximum(m[h], acc))
                return tuple(out)

            return lax.fori_loop(0, G, grp, m)

        fire_idx(0, 0)
        fire_idx(1, 1)
        pltpu.async_copy(a_h.at[pl.ds(0, C * 4)], ast0, semw0)
        pltpu.async_copy(a_h.at[pl.ds(0, C * 4)], ast1, semw1)
        wait_idx(0)
        fire_gather(0)

        def iter_pair(kk2, m):
            for b in (0, 1):
                kk = kk2 * 2 + b
                bp = 1 - b
                wait_idx(bp)
                fire_gather(bp)
                wait_gather(b)
                wait_w(b)
                m = compute(b, m)
                off = off_of(kk)
                pltpu.async_copy(bufs[b][4], a_h.at[pl.ds(off * 4, C * 4)],
                                 bufs[b][7])
                fire_idx(kk + 2, b)
            return m

        minit = tuple(jnp.full((LN,), -1e30, jnp.float32) for _ in range(4))
        m = lax.fori_loop(0, nitp // 2, iter_pair, minit)
        wait_idx(1)
        wait_gather(0)
        wait_w(0)
        wait_w(1)

        mv = jnp.full((LN,), -1e30, jnp.float32)
        for h in range(4):
            mv = jnp.where(lane == h, jnp.max(m[h]), mv)
        mst[...] = mv
        pltpu.sync_copy(mst, tmax.at[et, wid])


@functools.partial(jax.jit, static_argnames=("ep",))
def _passa(ep, q0, k0, s0, d0, q1, k1, s1, d1):
    nchunk = ep // C
    iters = -(-nchunk // (NC * NS))
    nitp = iters + (iters % 2)
    kern = pl.kernel(
        functools.partial(_passa_body, nchunk, nitp),
        out_type=(
            jax.ShapeDtypeStruct((ep * 4,), jnp.float32),
            jax.ShapeDtypeStruct((ep * 4,), jnp.float32),
            jax.ShapeDtypeStruct((2, NC * NS, LN), jnp.float32),
        ),
        mesh=_mesh(),
        scratch_types=(
            [pltpu.VMEM((C // 128, 128), jnp.int32),
             pltpu.VMEM((C // 128, 128), jnp.int32),
             pltpu.VMEM((C, 64), jnp.float32),
             pltpu.VMEM((C, 64), jnp.float32),
             pltpu.VMEM((C * 4,), jnp.float32)] * 2
            + [pltpu.VMEM((LN,), jnp.float32)]
            + [pltpu.SemaphoreType.DMA] * 6
        ),
        compiler_params=pltpu.CompilerParams(
            needs_layout_passes=False, use_tc_tiling_on_sc=False),
    )
    return kern(q0, k0, s0, d0, q1, k1, s1, d1)


def _passb_body(n_nodes, e_real, nchunk, rows_sp, rhalf, nitp,
                alpha, tmax, s_e, d_e, ve,
                numer, den,
                idxd0, idxs0, ldst0, vg0, al0, msg0,
                idxd1, idxs1, ldst1, vg1, al1, msg1,
                semi0, semi1, semg0, semg1, sems0, sems1, agg_sp):
    cc = lax.axis_index("c")
    sid = lax.axis_index("s")
    lane = lax.iota(jnp.int32, LN)
    quarter = n_nodes // 4
    base_g = (rhalf * NC + cc) * quarter
    base_l = cc * quarter
    trash = quarter
    G = C // LN
    NJ = C // 128
    zz = jnp.zeros((LN,), jnp.float32)
    tv = jnp.full((LN,), trash, jnp.int32)
    bufs = ((idxd0, idxs0, ldst0, vg0, al0, msg0, semi0, semg0, sems0),
            (idxd1, idxs1, ldst1, vg1, al1, msg1, semi1, semg1, sems1))

    pltpu.async_copy(tmax, msg0.at[pl.ds(0, NC * NS), pl.ds(0, LN)], semg0).wait()
    def mred(i, m):
        return jnp.maximum(m, msg0[i, pl.ds(0, LN)])
    m = lax.fori_loop(0, NC * NS, mred, jnp.full((LN,), -1e30, jnp.float32))
    lm = lane % 4
    gt = jnp.where(lm == 0, m[0],
                   jnp.where(lm == 1, m[1],
                             jnp.where(lm == 2, m[2], m[3])))

    for msg in (msg0, msg1):
        def zrow(e, _, msg=msg):
            for kcol in (0, 16, 32, 48, 56):
                msg[e, pl.ds(kcol, LN)] = zz
            return 0
        lax.fori_loop(0, C, zrow, 0)
    for ldst in (ldst0, ldst1):
        for j in range(NJ):
            for i in range(128 // LN):
                ldst[j, pl.ds(i * LN, LN)] = tv

    nz = -(-rows_sp // 128)
    def zchunk(kk, _):
        rr = jnp.minimum(kk * NS + sid, nz - 1)
        r0 = jnp.minimum(rr * 128, rows_sp - 128)
        pltpu.sync_copy(msg0.at[pl.ds(0, 128), :],
                        agg_sp.at[pl.ds(r0, 128), :])
        return 0
    lax.fori_loop(0, -(-nz // NS), zchunk, 0)
    plsc.subcore_barrier()

    def cid_of(kk):
        cidr = kk * NS + sid
        return cidr >= nchunk, pl.multiple_of(jnp.minimum(cidr, nchunk - 1) * C, 8)

    def fire_idx(kk, b):
        _, off = cid_of(kk)
        idxd, idxs, semi = bufs[b][0], bufs[b][1], bufs[b][6]
        for j in range(NJ):
            pltpu.async_copy(d_e.at[pl.ds(off + j * 128, 128)], idxd.at[j], semi)
            pltpu.async_copy(s_e.at[pl.ds(off + j * 128, 128)], idxs.at[j], semi)

    def wait_idx(b):
        idxd, idxs, semi = bufs[b][0], bufs[b][1], bufs[b][6]
        for j in range(NJ):
            pltpu.make_async_copy(d_e.at[pl.ds(0, 128)], idxd.at[j], semi).wait()
            pltpu.make_async_copy(s_e.at[pl.ds(0, 128)], idxs.at[j], semi).wait()

    def fire_gather(kk, b):
        _, off = cid_of(kk)
        idxs, vg, al, semg = bufs[b][1], bufs[b][3], bufs[b][4], bufs[b][7]
        pltpu.async_copy(alpha.at[pl.ds(off * 4, C * 4)], al, semg)
        for j in range(NJ):
            pltpu.async_copy(ve.at[idxs.at[j]], vg.at[pl.ds(j * 128, 128), :], semg)

    def wait_gather(b):
        idxs, vg, al, semg = bufs[b][1], bufs[b][3], bufs[b][4], bufs[b][7]
        pltpu.make_async_copy(alpha.at[pl.ds(0, C * 4)], al, semg).wait()
        for j in range(NJ):
            pltpu.make_async_copy(ve.at[idxs.at[j]], vg.at[pl.ds(j * 128, 128), :], semg).wait()

    def fire_scatter(b):
        ldst, msg, sems = bufs[b][2], bufs[b][5], bufs[b][8]
        for j in range(NJ):
            pltpu.async_copy(msg.at[pl.ds(j * 128, 128), :],
                             agg_sp.at[ldst.at[j]], sems, add=True)

    def wait_scatter(b):
        ldst, msg, sems = bufs[b][2], bufs[b][5], bufs[b][8]
        for j in range(NJ):
            pltpu.make_async_copy(msg.at[pl.ds(j * 128, 128), :],
                                  agg_sp.at[ldst.at[j]], sems).wait()

    def compute(kk, b):
        dup, off = cid_of(kk)
        idxd, ldst, vg, al, msg = (bufs[b][0], bufs[b][2], bufs[b][3],
                                   bufs[b][4], bufs[b][5])

        def exg(i, _):
            a = al[pl.ds(i * LN, LN)]
            al[pl.ds(i * LN, LN)] = jnp.exp(a - gt)
            return 0
        lax.fori_loop(0, C * 4 // LN, exg, 0)

        for j in range(NJ):
            for i in range(128 // LN):
                dd = idxd[j, pl.ds(i * LN, LN)]
                loc = dd - base_g
                eix = off + j * 128 + i * LN + lane
                ok = (loc >= 0) & (loc < quarter) & (eix < e_real) & jnp.logical_not(dup)
                ldst[j, pl.ds(i * LN, LN)] = jnp.where(ok, loc, trash)

        def grp(g, _):
            rows = g * LN + lane
            for h in range(4):
                exh = plsc.load_gather(al, [rows * 4 + h])
                plsc.store_scatter(msg, [rows, jnp.full((LN,), 64 + h, jnp.int32)], exh)
                for dd in range(16):
                    col = jnp.full((LN,), h * 16 + dd, jnp.int32)
                    vv = plsc.load_gather(vg, [rows, col])
                    plsc.store_scatter(msg, [rows, col], vv * exh)
            return 0
        lax.fori_loop(0, G, grp, 0)

    fire_idx(0, 0)
    fire_idx(1, 1)
    fire_scatter(0)
    fire_scatter(1)
    wait_idx(0)
    fire_gather(0, 0)

    def iter_pair(kk2, _):
        for b in (0, 1):
            kk = kk2 * 2 + b
            bp = 1 - b
            wait_idx(bp)
            fire_gather(kk + 1, bp)
            wait_gather(b)
            wait_scatter(b)
            compute(kk, b)
            fire_scatter(b)
            fire_idx(kk + 2, b)
        return 0

    lax.fori_loop(0, nitp // 2, iter_pair, 0)
    wait_idx(1)
    wait_gather(0)
    wait_scatter(0)
    wait_scatter(1)
    plsc.subcore_barrier()

    nf = quarter // FB
    def flush(kk, _):
        rr = jnp.minimum(kk * NS + sid, nf - 1)
        r0 = rr * FB
        pltpu.sync_copy(agg_sp.at[pl.ds(r0, FB), pl.ds(0, 64)],
                        numer.at[pl.ds(base_l + r0, FB), :])
        pltpu.sync_copy(agg_sp.at[pl.ds(r0, FB), pl.ds(64, 8)],
                        den.at[pl.ds(base_l + r0, FB), :])
        return 0
    lax.fori_loop(0, -(-nf // NS), flush, 0)


def _passb_half(n_nodes, e_real, ep, rhalf, alpha, tmax, s_e, d_e, ve):
    nchunk = ep // C
    rows_sp = n_nodes // 4 + 8
    iters = -(-nchunk // NS)
    nitp = iters + (iters % 2)
    kern = pl.kernel(
        functools.partial(_passb_body, n_nodes, e_real, nchunk, rows_sp,
                          rhalf, nitp),
        out_type=(
            jax.ShapeDtypeStruct((n_nodes // 2, 64), jnp.float32),
            jax.ShapeDtypeStruct((n_nodes // 2, 8), jnp.float32),
        ),
        mesh=_mesh(),
        scratch_types=(
            [pltpu.VMEM((C // 128, 128), jnp.int32),
             pltpu.VMEM((C // 128, 128), jnp.int32),
             pltpu.VMEM((C // 128, 128), jnp.int32),
             pltpu.VMEM((C, 64), jnp.float32),
             pltpu.VMEM((C * 4,), jnp.float32),
             pltpu.VMEM((C, 72), jnp.float32)] * 2
            + [pltpu.SemaphoreType.DMA] * 6
            + [pltpu.VMEM_SHARED((rows_sp, 72), jnp.float32)]
        ),
        compiler_params=pltpu.CompilerParams(
            needs_layout_passes=False, use_tc_tiling_on_sc=False),
    )
    return kern(alpha, tmax, s_e, d_e, ve)


@functools.partial(jax.jit, static_argnames=("n_nodes", "e_real", "ep"))
def _passb(n_nodes, e_real, ep, alpha, tmax, s_e, d_e, ve):
    n0, d0 = _passb_half(n_nodes, e_real, ep, 0, alpha, tmax, s_e, d_e, ve)
    n1, d1 = _passb_half(n_nodes, e_real, ep, 1, alpha, tmax, s_e, d_e, ve)
    return (jnp.concatenate([n0, n1], axis=0),
            jnp.concatenate([d0, d1], axis=0))


def _block_rows(m):
    for bm in (2000, 1000, 400, 200, 80, 40, 16, 8):
        if m % bm == 0:
            return bm
    return m


def _mm_body(x_ref, w_ref, b_ref, o_ref):
    o_ref[...] = (
        jnp.dot(x_ref[...], w_ref[...], preferred_element_type=jnp.float32)
        + b_ref[...]
    )


def _mm_bias(x, w, b):
    m, kd = x.shape
    n = w.shape[1]
    bm = _block_rows(m)
    return pl.pallas_call(
        _mm_body,
        grid=(m // bm,),
        in_specs=[
            pl.BlockSpec((bm, kd), lambda i: (i, 0)),
            pl.BlockSpec((kd, n), lambda i: (0, 0)),
            pl.BlockSpec((1, n), lambda i: (0, 0)),
        ],
        out_specs=pl.BlockSpec((bm, n), lambda i: (i, 0)),
        out_shape=jax.ShapeDtypeStruct((m, n), jnp.float32),
    )(x, w, b.reshape(1, n))


def _kqv_body(h_ref, w_ref, b_ref, k_ref, q_ref, v_ref):
    hh = h_ref[...]
    for i, o_ref in enumerate((k_ref, q_ref, v_ref)):
        o_ref[...] = (
            jnp.dot(hh, w_ref[i], preferred_element_type=jnp.float32)
            + b_ref[0, i]
        )


def _kqv(h, w3, b3):
    m = h.shape[0]
    bm = _block_rows(m)
    out = jax.ShapeDtypeStruct((m, 64), jnp.float32)
    return pl.pallas_call(
        _kqv_body,
        grid=(m // bm,),
        in_specs=[
            pl.BlockSpec((bm, 64), lambda i: (i, 0)),
            pl.BlockSpec((3, 64, 64), lambda i: (0, 0, 0)),
            pl.BlockSpec((1, 3, 64), lambda i: (0, 0, 0)),
        ],
        out_specs=[pl.BlockSpec((bm, 64), lambda i: (i, 0))] * 3,
        out_shape=[out, out, out],
    )(h, w3, b3.reshape(1, 3, 64))


def _post_body(numer_ref, den_ref, h_ref, w_ref, p_ref, o_ref):
    s = (lax.broadcasted_iota(jnp.int32, (8, 64), 1) // 16
         == lax.broadcasted_iota(jnp.int32, (8, 64), 0)).astype(jnp.float32)
    den_b = jnp.dot(den_ref[...], s, preferred_element_type=jnp.float32)
    agg = numer_ref[...] / (den_b + 1e-16)
    o = jax.nn.gelu(agg)
    o = (jnp.dot(o, w_ref[...], preferred_element_type=jnp.float32)
         + p_ref[0:1, :] + p_ref[1:2, :] * h_ref[...])
    mu = jnp.mean(o, axis=1, keepdims=True)
    var = jnp.mean((o - mu) ** 2, axis=1, keepdims=True)
    o = (o - mu) / jnp.sqrt(var + 1e-5) * p_ref[2:3, :] + p_ref[3:4, :]
    o_ref[...] = jax.nn.gelu(o)


def _post(numer, den, h, w_eff, pvec):
    m = h.shape[0]
    bm = _block_rows(m)
    return pl.pallas_call(
        _post_body,
        grid=(m // bm,),
        in_specs=[
            pl.BlockSpec((bm, 64), lambda i: (i, 0)),
            pl.BlockSpec((bm, 8), lambda i: (i, 0)),
            pl.BlockSpec((bm, 64), lambda i: (i, 0)),
            pl.BlockSpec((64, 64), lambda i: (0, 0)),
            pl.BlockSpec((4, 64), lambda i: (0, 0)),
        ],
        out_specs=pl.BlockSpec((bm, 64), lambda i: (i, 0)),
        out_shape=jax.ShapeDtypeStruct((m, 64), jnp.float32),
    )(numer, den, h, w_eff, pvec)


def kernel(x_user, x_item, ei_ui, ei_iu, W_in, b_in, kqva_W, kqva_b,
           rel_a, rel_m, rel_p, skip, ln_g, ln_b, W_out, b_out):
    n_u, n_i = x_user.shape[0], x_item.shape[0]
    e = ei_ui.shape[1]
    ep = -(-e // C) * C
    L = kqva_W.shape[0]

    pad = jnp.zeros((ep - e,), jnp.int32) if ep != e else None
    def _pad(a):
        a = a.astype(jnp.int32)
        return a if pad is None else jnp.concatenate([a, pad])
    s0, d0 = _pad(ei_ui[0]), _pad(ei_ui[1])
    s1, d1 = _pad(ei_iu[0]), _pad(ei_iu[1])

    h = [_mm_bias(x_user, W_in[0], b_in[0]), _mm_bias(x_item, W_in[1], b_in[1])]

    for l in range(L):
        kqv = []
        for t in range(2):
            A, M = rel_a[l, t], rel_m[l, t]
            scale = jnp.repeat(rel_p[l, t] / 4.0, 16)
            wk = jnp.einsum("dhe,hef->dhf",
                            kqva_W[l, 0, t].reshape(64, 4, 16), A).reshape(64, 64)
            bk = jnp.einsum("he,hef->hf",
                            kqva_b[l, 0, t].reshape(4, 16), A).reshape(64)
            wv = jnp.einsum("dhe,hef->dhf",
                            kqva_W[l, 2, t].reshape(64, 4, 16), M).reshape(64, 64)
            bv = jnp.einsum("he,hef->hf",
                            kqva_b[l, 2, t].reshape(4, 16), M).reshape(64)
            w3 = jnp.stack([wk * scale[None, :], kqva_W[l, 1, t], wv])
            b3 = jnp.stack([bk * scale, kqva_b[l, 1, t], bv])
            kqv.append(_kqv(h[t], w3, b3))
        (k0, q0, v0), (k1, q1, v1) = kqv

        alpha0, alpha1, tmax = _passa(ep, q1, k0, s0, d0, q0, k1, s1, d1)
        numer1, den1 = _passb(n_i, e, ep, alpha0, tmax[0], s0, d0, v0)
        numer0, den0 = _passb(n_u, e, ep, alpha1, tmax[1], s1, d1, v1)

        new_h = []
        for t, (numer, den) in enumerate(((numer0, den0), (numer1, den1))):
            beta = jax.nn.sigmoid(skip[l, t])
            pvec = jnp.stack([
                kqva_b[l, 3, t] * beta,
                jnp.full((64,), 1.0, jnp.float32) * (1.0 - beta),
                ln_g[l, t],
                ln_b[l, t],
            ])
            new_h.append(_post(numer, den, h[t], kqva_W[l, 3, t] * beta, pvec))
        h = new_h

    return (_mm_bias(h[0], W_out[0], b_out[0]),
            _mm_bias(h[1], W_out[1], b_out[1]))


# speedup vs baseline: 20.8959x; 1.2614x over previous
import functools
import math

import jax
import jax.numpy as jnp
from jax import lax
from jax.experimental import pallas as pl
from jax.experimental.pallas import tpu as pltpu
from jax.experimental.pallas import tpu_sc as plsc

NC = 2
NS = 16
LN = 16
C = 112
FB = 250


def _mesh():
    return plsc.VectorSubcoreMesh(
        core_axis_name="c", subcore_axis_name="s", num_cores=NC, num_subcores=NS
    )


def _passa_body(nchunk, nitp,
                q0, k0, s0, d0, q1, k1, s1, d1,
                a0, a1, tmax,
                idxd0, idxs0, qg0, kg0, ast0,
                idxd1, idxs1, qg1, kg1, ast1, mst,
                semi0, semi1, semg0, semg1, semw0, semw1):
    cc = lax.axis_index("c")
    ss = lax.axis_index("s")
    wid = ss * NC + cc
    lane = lax.iota(jnp.int32, LN)
    G = C // LN
    NJ = C // 128
    bufs = ((idxd0, idxs0, qg0, kg0, ast0, semi0, semg0, semw0),
            (idxd1, idxs1, qg1, kg1, ast1, semi1, semg1, semw1))

    for et, (q_h, k_h, s_h, d_h, a_h) in enumerate(
            ((q0, k0, s0, d0, a0), (q1, k1, s1, d1, a1))):

        def off_of(kk):
            cid = jnp.minimum(kk * (NC * NS) + wid, nchunk - 1)
            return pl.multiple_of(cid * C, 8)

        def fire_idx(kk, b):
            off = off_of(kk)
            idxd, idxs, semi = bufs[b][0], bufs[b][1], bufs[b][5]
            pltpu.async_copy(d_h.at[pl.ds(off, C)], idxd.at[0], semi)
            pltpu.async_copy(s_h.at[pl.ds(off, C)], idxs.at[0], semi)

        def wait_idx(b):
            idxd, idxs, semi = bufs[b][0], bufs[b][1], bufs[b][5]
            pltpu.make_async_copy(d_h.at[pl.ds(0, C)], idxd.at[0], semi).wait()
            pltpu.make_async_copy(s_h.at[pl.ds(0, C)], idxs.at[0], semi).wait()

        def fire_gather(b):
            idxd, idxs, qg, kg, semg = (bufs[b][0], bufs[b][1], bufs[b][2],
                                        bufs[b][3], bufs[b][6])
            pltpu.async_copy(q_h.at[idxd.at[0]], qg, semg)
            pltpu.async_copy(k_h.at[idxs.at[0]], kg, semg)

        def wait_gather(b):
            idxd, idxs, qg, kg, semg = (bufs[b][0], bufs[b][1], bufs[b][2],
                                        bufs[b][3], bufs[b][6])
            pltpu.make_async_copy(q_h.at[idxd.at[0]], qg, semg).wait()
            pltpu.make_async_copy(k_h.at[idxs.at[0]], kg, semg).wait()

        def wait_w(b):
            ast, semw = bufs[b][4], bufs[b][7]
            pltpu.make_async_copy(a_h.at[pl.ds(0, C * 4)], ast, semw).wait()

        def compute(b, m):
            qg, kg, ast = bufs[b][2], bufs[b][3], bufs[b][4]

            def grp(g, m):
                rows = g * LN + lane
                out = []
                for h in range(4):
                    acc = jnp.zeros((LN,), jnp.float32)
                    for dd in range(16):
                        col = jnp.full((LN,), h * 16 + dd, jnp.int32)
                        vq = plsc.load_gather(qg, [rows, col])
                        vk = plsc.load_gather(kg, [rows, col])
                        acc = acc + vq * vk
                    plsc.store_scatter(ast, [rows * 4 + h], acc)
                    out.append(jnp.maximum(m[h], acc))
                return tuple(out)

            return lax.fori_loop(0, G, grp, m)

        fire_idx(0, 0)
        fire_idx(1, 1)
        pltpu.async_copy(a_h.at[pl.ds(0, C * 4)], ast0, semw0)
        pltpu.async_copy(a_h.at[pl.ds(0, C * 4)], ast1, semw1)
        wait_idx(0)
        fire_gather(0)

        def iter_pair(kk2, m):
            for b in (0, 1):
                kk = kk2 * 2 + b
                bp = 1 - b
                wait_idx(bp)
                fire_gather(bp)
                wait_gather(b)
                wait_w(b)
                m = compute(b, m)
                off = off_of(kk)
                pltpu.async_copy(bufs[b][4], a_h.at[pl.ds(off * 4, C * 4)],
                                 bufs[b][7])
                fire_idx(kk + 2, b)
            return m

        minit = tuple(jnp.full((LN,), -1e30, jnp.float32) for _ in range(4))
        m = lax.fori_loop(0, nitp // 2, iter_pair, minit)
        wait_idx(1)
        wait_gather(0)
        wait_w(0)
        wait_w(1)

        mv = jnp.full((LN,), -1e30, jnp.float32)
        for h in range(4):
            mv = jnp.where(lane == h, jnp.max(m[h]), mv)
        mst[...] = mv
        pltpu.sync_copy(mst, tmax.at[et, wid])


@functools.partial(jax.jit, static_argnames=("ep",))
def _passa(ep, q0, k0, s0, d0, q1, k1, s1, d1):
    nchunk = ep // C
    iters = -(-nchunk // (NC * NS))
    nitp = iters + (iters % 2)
    kern = pl.kernel(
        functools.partial(_passa_body, nchunk, nitp),
        out_type=(
            jax.ShapeDtypeStruct((ep * 4,), jnp.float32),
            jax.ShapeDtypeStruct((ep * 4,), jnp.float32),
            jax.ShapeDtypeStruct((2, NC * NS, LN), jnp.float32),
        ),
        mesh=_mesh(),
        scratch_types=(
            [pltpu.VMEM((1, C), jnp.int32),
             pltpu.VMEM((1, C), jnp.int32),
             pltpu.VMEM((C, 64), jnp.float32),
             pltpu.VMEM((C, 64), jnp.float32),
             pltpu.VMEM((C * 4,), jnp.float32)] * 2
            + [pltpu.VMEM((LN,), jnp.float32)]
            + [pltpu.SemaphoreType.DMA] * 6
        ),
        compiler_params=pltpu.CompilerParams(
            needs_layout_passes=False, use_tc_tiling_on_sc=False),
    )
    return kern(q0, k0, s0, d0, q1, k1, s1, d1)


def _passb_body(n_nodes, e_real, nchunk, rows_sp, nitp,
                alpha, tmax, s_e, d_e, ve,
                numer, den,
                idxd0, idxs0, ldst0, al0, msg0, exd0,
                idxd1, idxs1, ldst1, al1, msg1, exd1,
                semi0, semi1, semg0, semg1, sems0, sems1, agg_sp, den_sp):
    cc = lax.axis_index("c")
    sid = lax.axis_index("s")
    lane = lax.iota(jnp.int32, LN)
    half = n_nodes // 2
    base = cc * half
    trash = half
    G = C // LN
    zz = jnp.zeros((LN,), jnp.float32)
    tv = jnp.full((LN,), trash, jnp.int32)
    bufs = ((idxd0, idxs0, ldst0, al0, msg0, exd0, semi0, semg0, sems0),
            (idxd1, idxs1, ldst1, al1, msg1, exd1, semi1, semg1, sems1))

    pltpu.async_copy(tmax, msg0.at[pl.ds(0, NC * NS), pl.ds(0, LN)], semg0).wait()
    def mred(i, m):
        return jnp.maximum(m, msg0[i, pl.ds(0, LN)])
    m = lax.fori_loop(0, NC * NS, mred, jnp.full((LN,), -1e30, jnp.float32))
    lm = lane % 4
    gt = jnp.where(lm == 0, m[0],
                   jnp.where(lm == 1, m[1],
                             jnp.where(lm == 2, m[2], m[3])))

    for msg in (msg0, msg1):
        def zrow(e, _, msg=msg):
            for kcol in range(0, 64, LN):
                msg[e, pl.ds(kcol, LN)] = zz
            return 0
        lax.fori_loop(0, C, zrow, 0)
    for exd in (exd0, exd1):
        def zex3(g, _, exd=exd):
            rows = g * LN + lane
            for col in range(8):
                plsc.store_scatter(exd, [rows, jnp.full((LN,), col, jnp.int32)], zz)
            return 0
        lax.fori_loop(0, G, zex3, 0)
    for ldst in (ldst0, ldst1):
        for i in range(C // LN):
            ldst[0, pl.ds(i * LN, LN)] = tv

    nz = -(-rows_sp // C)
    def zchunk(kk, _):
        rr = jnp.minimum(kk * NS + sid, nz - 1)
        r0 = jnp.minimum(rr * C, rows_sp - C)
        pltpu.sync_copy(msg0.at[pl.ds(0, C), :], agg_sp.at[pl.ds(r0, C), :])
        pltpu.sync_copy(exd0, den_sp.at[pl.ds(r0, C), :])
        return 0
    lax.fori_loop(0, -(-nz // NS), zchunk, 0)
    plsc.subcore_barrier()

    def cid_of(kk):
        cidr = kk * NS + sid
        return cidr >= nchunk, pl.multiple_of(jnp.minimum(cidr, nchunk - 1) * C, 8)

    def fire_idx(kk, b):
        _, off = cid_of(kk)
        idxd, idxs, semi = bufs[b][0], bufs[b][1], bufs[b][6]
        pltpu.async_copy(d_e.at[pl.ds(off, C)], idxd.at[0], semi)
        pltpu.async_copy(s_e.at[pl.ds(off, C)], idxs.at[0], semi)

    def wait_idx(b):
        idxd, idxs, semi = bufs[b][0], bufs[b][1], bufs[b][6]
        pltpu.make_async_copy(d_e.at[pl.ds(0, C)], idxd.at[0], semi).wait()
        pltpu.make_async_copy(s_e.at[pl.ds(0, C)], idxs.at[0], semi).wait()

    def fire_gather(kk, b):
        _, off = cid_of(kk)
        idxs, al, msg, semg = bufs[b][1], bufs[b][3], bufs[b][4], bufs[b][7]
        pltpu.async_copy(alpha.at[pl.ds(off * 4, C * 4)], al, semg)
        pltpu.async_copy(ve.at[idxs.at[0]], msg, semg)

    def wait_gather(b):
        idxs, al, msg, semg = bufs[b][1], bufs[b][3], bufs[b][4], bufs[b][7]
        pltpu.make_async_copy(alpha.at[pl.ds(0, C * 4)], al, semg).wait()
        pltpu.make_async_copy(ve.at[idxs.at[0]], msg, semg).wait()

    def fire_scatter(b):
        ldst, msg, exd, sems = bufs[b][2], bufs[b][4], bufs[b][5], bufs[b][8]
        pltpu.async_copy(msg, agg_sp.at[ldst.at[0]], sems, add=True)
        pltpu.async_copy(exd, den_sp.at[ldst.at[0]], sems, add=True)

    def wait_scatter(b):
        ldst, msg, exd, sems = bufs[b][2], bufs[b][4], bufs[b][5], bufs[b][8]
        pltpu.make_async_copy(msg, agg_sp.at[ldst.at[0]], sems).wait()
        pltpu.make_async_copy(exd, den_sp.at[ldst.at[0]], sems).wait()

    def compute(kk, b):
        dup, off = cid_of(kk)
        idxd, ldst, al, msg, exd = (bufs[b][0], bufs[b][2], bufs[b][3],
                                    bufs[b][4], bufs[b][5])

        def exg(i, _):
            a = al[pl.ds(i * LN, LN)]
            al[pl.ds(i * LN, LN)] = jnp.exp(a - gt)
            return 0
        lax.fori_loop(0, C * 4 // LN, exg, 0)

        for i in range(C // LN):
            dd = idxd[0, pl.ds(i * LN, LN)]
            loc = dd - base
            eix = off + i * LN + lane
            ok = (loc >= 0) & (loc < half) & (eix < e_real) & jnp.logical_not(dup)
            ldst[0, pl.ds(i * LN, LN)] = jnp.where(ok, loc, trash)

        def grp(g, _):
            rows = g * LN + lane
            for h in range(4):
                exh = plsc.load_gather(al, [rows * 4 + h])
                plsc.store_scatter(exd, [rows, jnp.full((LN,), h, jnp.int32)], exh)
                for dd in range(16):
                    col = jnp.full((LN,), h * 16 + dd, jnp.int32)
                    vv = plsc.load_gather(msg, [rows, col])
                    plsc.store_scatter(msg, [rows, col], vv * exh)
            return 0
        lax.fori_loop(0, G, grp, 0)

    fire_idx(0, 0)
    fire_idx(1, 1)
    fire_scatter(0)
    fire_scatter(1)
    wait_idx(0)
    fire_gather(0, 0)

    def iter_pair(kk2, _):
        for b in (0, 1):
            kk = kk2 * 2 + b
            bp = 1 - b
            wait_idx(bp)
            fire_gather(kk + 1, bp)
            wait_gather(b)
            wait_scatter(b)
            compute(kk, b)
            fire_scatter(b)
            fire_idx(kk + 2, b)
        return 0

    lax.fori_loop(0, nitp // 2, iter_pair, 0)
    wait_idx(1)
    wait_gather(0)
    wait_scatter(0)
    wait_scatter(1)
    plsc.subcore_barrier()

    nf = half // FB
    def flush(kk, _):
        rr = jnp.minimum(kk * NS + sid, nf - 1)
        r0 = rr * FB
        pltpu.sync_copy(agg_sp.at[pl.ds(r0, FB), :],
                        numer.at[pl.ds(base + r0, FB), :])
        pltpu.sync_copy(den_sp.at[pl.ds(r0, FB), :],
                        den.at[pl.ds(base + r0, FB), :])
        return 0
    lax.fori_loop(0, -(-nf // NS), flush, 0)


@functools.partial(jax.jit, static_argnames=("n_nodes", "e_real", "ep"))
def _passb(n_nodes, e_real, ep, alpha, tmax, s_e, d_e, ve):
    nchunk = ep // C
    rows_sp = n_nodes // 2 + 8
    iters = -(-nchunk // NS)
    nitp = iters + (iters % 2)
    kern = pl.kernel(
        functools.partial(_passb_body, n_nodes, e_real, nchunk, rows_sp, nitp),
        out_type=(
            jax.ShapeDtypeStruct((n_nodes, 64), jnp.float32),
            jax.ShapeDtypeStruct((n_nodes, 8), jnp.float32),
        ),
        mesh=_mesh(),
        scratch_types=(
            [pltpu.VMEM((1, C), jnp.int32),
             pltpu.VMEM((1, C), jnp.int32),
             pltpu.VMEM((1, C), jnp.int32),
             pltpu.VMEM((C * 4,), jnp.float32),
             pltpu.VMEM((C, 64), jnp.float32),
             pltpu.VMEM((C, 8), jnp.float32)] * 2
            + [pltpu.SemaphoreType.DMA] * 6
            + [pltpu.VMEM_SHARED((rows_sp, 64), jnp.float32),
               pltpu.VMEM_SHARED((rows_sp, 8), jnp.float32)]
        ),
        compiler_params=pltpu.CompilerParams(
            needs_layout_passes=False, use_tc_tiling_on_sc=False),
    )
    return kern(alpha, tmax, s_e, d_e, ve)


def _block_rows(m):
    for bm in (2000, 1000, 400, 200, 80, 40, 16, 8):
        if m % bm == 0:
            return bm
    return m


def _mm_body(x_ref, w_ref, b_ref, o_ref):
    o_ref[...] = (
        jnp.dot(x_ref[...], w_ref[...], preferred_element_type=jnp.float32)
        + b_ref[...]
    )


def _mm_bias(x, w, b):
    m, kd = x.shape
    n = w.shape[1]
    bm = _block_rows(m)
    return pl.pallas_call(
        _mm_body,
        grid=(m // bm,),
        in_specs=[
            pl.BlockSpec((bm, kd), lambda i: (i, 0)),
            pl.BlockSpec((kd, n), lambda i: (0, 0)),
            pl.BlockSpec((1, n), lambda i: (0, 0)),
        ],
        out_specs=pl.BlockSpec((bm, n), lambda i: (i, 0)),
        out_shape=jax.ShapeDtypeStruct((m, n), jnp.float32),
    )(x, w, b.reshape(1, n))


def _kqv_body(h_ref, w_ref, b_ref, k_ref, q_ref, v_ref):
    hh = h_ref[...]
    for i, o_ref in enumerate((k_ref, q_ref, v_ref)):
        o_ref[...] = (
            jnp.dot(hh, w_ref[i], preferred_element_type=jnp.float32)
            + b_ref[0, i]
        )


def _kqv(h, w3, b3):
    m = h.shape[0]
    bm = _block_rows(m)
    out = jax.ShapeDtypeStruct((m, 64), jnp.float32)
    return pl.pallas_call(
        _kqv_body,
        grid=(m // bm,),
        in_specs=[
            pl.BlockSpec((bm, 64), lambda i: (i, 0)),
            pl.BlockSpec((3, 64, 64), lambda i: (0, 0, 0)),
            pl.BlockSpec((1, 3, 64), lambda i: (0, 0, 0)),
        ],
        out_specs=[pl.BlockSpec((bm, 64), lambda i: (i, 0))] * 3,
        out_shape=[out, out, out],
    )(h, w3, b3.reshape(1, 3, 64))


def _post_body(numer_ref, den_ref, h_ref, w_ref, p_ref, o_ref):
    s = (lax.broadcasted_iota(jnp.int32, (8, 64), 1) // 16
         == lax.broadcasted_iota(jnp.int32, (8, 64), 0)).astype(jnp.float32)
    den_b = jnp.dot(den_ref[...], s, preferred_element_type=jnp.float32)
    agg = numer_ref[...] / (den_b + 1e-16)
    o = jax.nn.gelu(agg)
    o = (jnp.dot(o, w_ref[...], preferred_element_type=jnp.float32)
         + p_ref[0:1, :] + p_ref[1:2, :] * h_ref[...])
    mu = jnp.mean(o, axis=1, keepdims=True)
    var = jnp.mean((o - mu) ** 2, axis=1, keepdims=True)
    o = (o - mu) / jnp.sqrt(var + 1e-5) * p_ref[2:3, :] + p_ref[3:4, :]
    o_ref[...] = jax.nn.gelu(o)


def _post(numer, den, h, w_eff, pvec):
    m = h.shape[0]
    bm = _block_rows(m)
    return pl.pallas_call(
        _post_body,
        grid=(m // bm,),
        in_specs=[
            pl.BlockSpec((bm, 64), lambda i: (i, 0)),
            pl.BlockSpec((bm, 8), lambda i: (i, 0)),
            pl.BlockSpec((bm, 64), lambda i: (i, 0)),
            pl.BlockSpec((64, 64), lambda i: (0, 0)),
            pl.BlockSpec((4, 64), lambda i: (0, 0)),
        ],
        out_specs=pl.BlockSpec((bm, 64), lambda i: (i, 0)),
        out_shape=jax.ShapeDtypeStruct((m, 64), jnp.float32),
    )(numer, den, h, w_eff, pvec)


def kernel(x_user, x_item, ei_ui, ei_iu, W_in, b_in, kqva_W, kqva_b,
           rel_a, rel_m, rel_p, skip, ln_g, ln_b, W_out, b_out):
    n_u, n_i = x_user.shape[0], x_item.shape[0]
    e = ei_ui.shape[1]
    ep = -(-e // C) * C
    L = kqva_W.shape[0]

    pad = jnp.zeros((ep - e,), jnp.int32) if ep != e else None
    def _pad(a):
        a = a.astype(jnp.int32)
        return a if pad is None else jnp.concatenate([a, pad])
    s0, d0 = _pad(ei_ui[0]), _pad(ei_ui[1])
    s1, d1 = _pad(ei_iu[0]), _pad(ei_iu[1])

    h = [_mm_bias(x_user, W_in[0], b_in[0]), _mm_bias(x_item, W_in[1], b_in[1])]

    for l in range(L):
        kqv = []
        for t in range(2):
            A, M = rel_a[l, t], rel_m[l, t]
            scale = jnp.repeat(rel_p[l, t] / 4.0, 16)
            wk = jnp.einsum("dhe,hef->dhf",
                            kqva_W[l, 0, t].reshape(64, 4, 16), A).reshape(64, 64)
            bk = jnp.einsum("he,hef->hf",
                            kqva_b[l, 0, t].reshape(4, 16), A).reshape(64)
            wv = jnp.einsum("dhe,hef->dhf",
                            kqva_W[l, 2, t].reshape(64, 4, 16), M).reshape(64, 64)
            bv = jnp.einsum("he,hef->hf",
                            kqva_b[l, 2, t].reshape(4, 16), M).reshape(64)
            w3 = jnp.stack([wk * scale[None, :], kqva_W[l, 1, t], wv])
            b3 = jnp.stack([bk * scale, kqva_b[l, 1, t], bv])
            kqv.append(_kqv(h[t], w3, b3))
        (k0, q0, v0), (k1, q1, v1) = kqv

        alpha0, alpha1, tmax = _passa(ep, q1, k0, s0, d0, q0, k1, s1, d1)
        numer1, den1 = _passb(n_i, e, ep, alpha0, tmax[0], s0, d0, v0)
        numer0, den0 = _passb(n_u, e, ep, alpha1, tmax[1], s1, d1, v1)

        new_h = []
        for t, (numer, den) in enumerate(((numer0, den0), (numer1, den1))):
            beta = jax.nn.sigmoid(skip[l, t])
            pvec = jnp.stack([
                kqva_b[l, 3, t] * beta,
                jnp.full((64,), 1.0, jnp.float32) * (1.0 - beta),
                ln_g[l, t],
                ln_b[l, t],
            ])
            new_h.append(_post(numer, den, h[t], kqva_W[l, 3, t] * beta, pvec))
        h = new_h

    return (_mm_bias(h[0], W_out[0], b_out[0]),
            _mm_bias(h[1], W_out[1], b_out[1]))
